# baseline (speedup 1.0000x reference)
import jax
import jax.numpy as jnp
from jax import lax
from jax.experimental import pallas as pl
from jax.experimental.pallas import tpu as pltpu
from jax.experimental.pallas import tpu_sc as plsc

BATCH = 16384
HIST = 50
FEAT = 4
EMBED = 32
NROWS = BATCH * HIST
NC = 2
NS = 16
NW = NC * NS
BT = 128
NBT = BATCH // BT
BT_PER_W = NBT // NW
B_PER_W = BT_PER_W * BT
N_PER_W = B_PER_W * HIST
NET = EMBED // 8
GSLICE = 128
XCH = 5


def _body(xp_hbm, table_hbm, out_hbm, xbuf, idxbuf, rows, tbuf,
          sx0, sx1, sg0, sg1, sw0, sw1):
    wid = lax.axis_index("s") * NC + lax.axis_index("c")
    bt0 = wid * BT_PER_W
    lanes = lax.iota(jnp.int32, 16)

    def xchunk(s, carry):
        h0 = s * (2 * XCH)
        d0 = pltpu.async_copy(
            xp_hbm.at[pl.ds(h0, XCH), pl.ds(bt0, BT_PER_W)], xbuf.at[0], sx0)
        d1 = pltpu.async_copy(
            xp_hbm.at[pl.ds(h0 + XCH, XCH), pl.ds(bt0, BT_PER_W)],
            xbuf.at[1], sx1)

        for half, dma in ((0, d0), (1, d1)):
            dma.wait()

            def hl_loop(hl, carry2, half=half):
                h = h0 + half * XCH + hl
                for btl in range(BT_PER_W):

                    def grp(g, carry3, btl=btl, hl=hl, h=h, half=half):
                        p = g * 16
                        v0 = xbuf[half, hl, btl, pl.ds(p, 16)]
                        v1 = xbuf[half, hl, btl, pl.ds(p + BT, 16)]
                        v2 = xbuf[half, hl, btl, pl.ds(p + 2 * BT, 16)]
                        v3 = xbuf[half, hl, btl, pl.ds(p + 3 * BT, 16)]
                        s_ = (v0 + v2) + (v1 + v3)
                        idxv = (s_ * 0.25).astype(jnp.int32)
                        idxbuf[pl.ds(h * B_PER_W + btl * BT + g * 16, 16)] = idxv
                        return carry3

                    lax.fori_loop(0, BT // 16, grp, 0)
                return carry2

            lax.fori_loop(0, XCH, hl_loop, 0)
        return carry

    lax.fori_loop(0, HIST // (2 * XCH), xchunk, 0)

    def fire_gathers(h, buf, sem):
        return [
            pltpu.async_copy(
                table_hbm.at[idxbuf.at[pl.ds(h * B_PER_W + j * GSLICE, GSLICE)]],
                rows.at[buf, pl.ds(j * GSLICE, GSLICE)],
                sem,
            )
            for j in range(B_PER_W // GSLICE)
        ]

    def transpose(buf):
        def trans(g, carry2):
            b0 = g * 16
            b_idx = b0 + lanes
            btl_v = jnp.broadcast_to((b0 >> 7).astype(jnp.int32), (16,))
            bl_v = (b0 & 127) + lanes
            for e0 in range(EMBED):
                e_l = (e0 + lanes) & 31
                v = plsc.load_gather(rows.at[buf], [b_idx, e_l])
                plsc.store_scatter(
                    tbuf.at[buf],
                    [e_l >> 3, btl_v, ((e_l & 7) << 7) + bl_v],
                    v,
                )
            return carry2

        lax.fori_loop(0, B_PER_W // 16, trans, 0)

    def fire_writes(h, buf, sem):
        return [
            pltpu.async_copy(
                tbuf.at[buf, et],
                out_hbm.at[h, et, pl.ds(bt0, BT_PER_W)],
                sem,
            )
            for et in range(NET)
        ]

    def hstep(s, carry):
        h0 = 2 * s
        g0 = fire_gathers(h0, 0, sg0)
        g1 = fire_gathers(h0 + 1, 1, sg1)
        for cp in g0:
            cp.wait()
        transpose(0)
        w0 = fire_writes(h0, 0, sw0)
        for cp in g1:
            cp.wait()
        transpose(1)
        w1 = fire_writes(h0 + 1, 1, sw1)
        for cp in w0:
            cp.wait()
        for cp in w1:
            cp.wait()
        return carry

    lax.fori_loop(0, HIST // 2, hstep, 0)


VC = 800
NVC = 1000000 // VC


def _tbody(tT_hbm, out_hbm, vbuf, tbuf2, sg0, sg1, sw0, sw1):
    wid = lax.axis_index("s") * NC + lax.axis_index("c")
    lanes = lax.iota(jnp.int32, 16)

    def transpose(buf):
        def grp(g, carry):
            vl = g * 16 + lanes
            for e0 in range(EMBED):
                e_l = (e0 + lanes) & 31
                v = plsc.load_gather(vbuf.at[buf], [e_l, vl])
                plsc.store_scatter(tbuf2.at[buf], [vl, e_l], v)
            return carry

        lax.fori_loop(0, VC // 16, grp, 0)

    def step(s_, carry):
        t0 = wid + (2 * NW) * s_
        t1 = t0 + NW
        p1 = t1 < NVC
        d0 = pltpu.async_copy(
            tT_hbm.at[:, pl.ds(t0 * VC, VC)], vbuf.at[0], sg0)

        @pl.when(p1)
        def _():
            pltpu.async_copy(tT_hbm.at[:, pl.ds(t1 * VC, VC)], vbuf.at[1], sg1)

        d0.wait()
        transpose(0)
        w0 = pltpu.async_copy(tbuf2.at[0], out_hbm.at[pl.ds(t0 * VC, VC)], sw0)

        @pl.when(p1)
        def _():
            pltpu.make_async_copy(
                tT_hbm.at[:, pl.ds(t1 * VC, VC)], vbuf.at[1], sg1).wait()
            transpose(1)
            pltpu.async_copy(tbuf2.at[1], out_hbm.at[pl.ds(t1 * VC, VC)], sw1)

        w0.wait()

        @pl.when(p1)
        def _():
            pltpu.make_async_copy(
                tbuf2.at[1], out_hbm.at[pl.ds(t1 * VC, VC)], sw1).wait()

        return carry

    lax.fori_loop(0, NVC // (2 * NW) + 1, step, 0)


def kernel(x, table):
    xp = jnp.transpose(x.reshape(NBT, BT, HIST, FEAT), (2, 0, 3, 1))
    xp = xp.reshape(HIST, NBT, FEAT * BT)
    run1 = pl.kernel(
        _tbody,
        out_type=jax.ShapeDtypeStruct((1000000, EMBED), jnp.float32),
        mesh=plsc.VectorSubcoreMesh(core_axis_name="c", subcore_axis_name="s"),
        compiler_params=pltpu.CompilerParams(
            needs_layout_passes=False, use_tc_tiling_on_sc=False
        ),
        scratch_types=[
            pltpu.VMEM((2, EMBED, VC), jnp.float32),
            pltpu.VMEM((2, VC, EMBED), jnp.float32),
            pltpu.SemaphoreType.DMA,
            pltpu.SemaphoreType.DMA,
            pltpu.SemaphoreType.DMA,
            pltpu.SemaphoreType.DMA,
        ],
    )
    run = pl.kernel(
        _body,
        out_type=jax.ShapeDtypeStruct((HIST, NET, NBT, 8 * BT), jnp.float32),
        mesh=plsc.VectorSubcoreMesh(core_axis_name="c", subcore_axis_name="s"),
        compiler_params=pltpu.CompilerParams(
            needs_layout_passes=False, use_tc_tiling_on_sc=False
        ),
        scratch_types=[
            pltpu.VMEM((2, XCH, BT_PER_W, FEAT * BT), jnp.float32),
            pltpu.VMEM((N_PER_W,), jnp.int32),
            pltpu.VMEM((2, B_PER_W, EMBED), jnp.float32),
            pltpu.VMEM((2, NET, BT_PER_W, 8 * BT), jnp.float32),
            pltpu.SemaphoreType.DMA,
            pltpu.SemaphoreType.DMA,
            pltpu.SemaphoreType.DMA,
            pltpu.SemaphoreType.DMA,
            pltpu.SemaphoreType.DMA,
            pltpu.SemaphoreType.DMA,
        ],
    )
    table_rm = run1(jnp.transpose(table))
    out5 = run(xp, table_rm)
    out = jnp.transpose(out5.reshape(HIST, NET, NBT, 8, BT), (2, 4, 0, 1, 3))
    return out.reshape(BATCH, HIST, EMBED)


# speedup vs baseline: 3.8780x; 3.8780x over previous
import jax
import jax.numpy as jnp
from jax import lax
from jax.experimental import pallas as pl
from jax.experimental.pallas import tpu as pltpu
from jax.experimental.pallas import tpu_sc as plsc

BATCH = 16384
HIST = 50
FEAT = 4
EMBED = 32
NROWS = BATCH * HIST
NC = 2
NS = 16
NW = NC * NS
BT = 128
NBT = BATCH // BT
BT_PER_W = NBT // NW
B_PER_W = BT_PER_W * BT
N_PER_W = B_PER_W * HIST
NET = EMBED // 8
GSLICE = 128
XCH = 5


def _body(xp_hbm, table_hbm, out_hbm, xbuf, idxbuf, rows, tbuf,
          sx0, sx1, sg0, sg1, sw0, sw1):
    wid = lax.axis_index("s") * NC + lax.axis_index("c")
    bt0 = wid * BT_PER_W
    lanes = lax.iota(jnp.int32, 16)

    def xchunk(s, carry):
        h0 = s * (2 * XCH)
        d0 = pltpu.async_copy(
            xp_hbm.at[pl.ds(h0, XCH), pl.ds(bt0, BT_PER_W)], xbuf.at[0], sx0)
        d1 = pltpu.async_copy(
            xp_hbm.at[pl.ds(h0 + XCH, XCH), pl.ds(bt0, BT_PER_W)],
            xbuf.at[1], sx1)

        for half, dma in ((0, d0), (1, d1)):
            dma.wait()

            def hl_loop(hl, carry2, half=half):
                h = h0 + half * XCH + hl
                for btl in range(BT_PER_W):

                    def grp(g, carry3, btl=btl, hl=hl, h=h, half=half):
                        p = g * 16
                        v0 = xbuf[half, hl, btl, pl.ds(p, 16)]
                        v1 = xbuf[half, hl, btl, pl.ds(p + BT, 16)]
                        v2 = xbuf[half, hl, btl, pl.ds(p + 2 * BT, 16)]
                        v3 = xbuf[half, hl, btl, pl.ds(p + 3 * BT, 16)]
                        s_ = (v0 + v2) + (v1 + v3)
                        idxv = (s_ * 0.25).astype(jnp.int32)
                        idxbuf[pl.ds(h * B_PER_W + btl * BT + g * 16, 16)] = idxv
                        return carry3

                    lax.fori_loop(0, BT // 16, grp, 0)
                return carry2

            lax.fori_loop(0, XCH, hl_loop, 0)
        return carry

    lax.fori_loop(0, HIST // (2 * XCH), xchunk, 0)

    def fire_gathers(h, buf, sem):
        return [
            pltpu.async_copy(
                table_hbm.at[idxbuf.at[pl.ds(h * B_PER_W + j * GSLICE, GSLICE)]],
                rows.at[buf, pl.ds(j * GSLICE, GSLICE)],
                sem,
            )
            for j in range(B_PER_W // GSLICE)
        ]

    def transpose(buf):
        def trans(g, carry2):
            b0 = g * 16
            b_idx = b0 + lanes
            btl_v = jnp.broadcast_to((b0 >> 7).astype(jnp.int32), (16,))
            bl_v = (b0 & 127) + lanes
            for e0 in range(EMBED):
                e_l = (e0 + lanes) & 31
                v = plsc.load_gather(rows.at[buf], [b_idx, e_l])
                plsc.store_scatter(
                    tbuf.at[buf],
                    [e_l >> 3, btl_v, ((e_l & 7) << 7) + bl_v],
                    v,
                )
            return carry2

        lax.fori_loop(0, B_PER_W // 16, trans, 0)

    def fire_writes(h, buf, sem):
        return [
            pltpu.async_copy(
                tbuf.at[buf, et],
                out_hbm.at[h, et, pl.ds(bt0, BT_PER_W)],
                sem,
            )
            for et in range(NET)
        ]

    def hstep(s, carry):
        h0 = 2 * s
        g0 = fire_gathers(h0, 0, sg0)
        g1 = fire_gathers(h0 + 1, 1, sg1)
        for cp in g0:
            cp.wait()
        transpose(0)
        w0 = fire_writes(h0, 0, sw0)
        for cp in g1:
            cp.wait()
        transpose(1)
        w1 = fire_writes(h0 + 1, 1, sw1)
        for cp in w0:
            cp.wait()
        for cp in w1:
            cp.wait()
        return carry

    lax.fori_loop(0, HIST // 2, hstep, 0)


def kernel(x, table):
    xp = jnp.transpose(x.reshape(NBT, BT, HIST, FEAT), (2, 0, 3, 1))
    xp = xp.reshape(HIST, NBT, FEAT * BT)
    run = pl.kernel(
        _body,
        out_type=jax.ShapeDtypeStruct((HIST, NET, NBT, 8 * BT), jnp.float32),
        mesh=plsc.VectorSubcoreMesh(core_axis_name="c", subcore_axis_name="s"),
        compiler_params=pltpu.CompilerParams(
            needs_layout_passes=False, use_tc_tiling_on_sc=False
        ),
        scratch_types=[
            pltpu.VMEM((2, XCH, BT_PER_W, FEAT * BT), jnp.float32),
            pltpu.VMEM((N_PER_W,), jnp.int32),
            pltpu.VMEM((2, B_PER_W, EMBED), jnp.float32),
            pltpu.VMEM((2, NET, BT_PER_W, 8 * BT), jnp.float32),
            pltpu.SemaphoreType.DMA,
            pltpu.SemaphoreType.DMA,
            pltpu.SemaphoreType.DMA,
            pltpu.SemaphoreType.DMA,
            pltpu.SemaphoreType.DMA,
            pltpu.SemaphoreType.DMA,
        ],
    )
    out5 = run(xp, table)
    out = jnp.transpose(out5.reshape(HIST, NET, NBT, 8, BT), (2, 4, 0, 1, 3))
    return out.reshape(BATCH, HIST, EMBED)
